# baseline (speedup 1.0000x reference)
import functools

import jax
import jax.numpy as jnp
from jax import lax
from jax.experimental import pallas as pl
from jax.experimental.pallas import tpu as pltpu
from jax.experimental.pallas import tpu_sc as plsc

B = 4096
L = 200
EMBED = 64
NC = 2
NS = 16
NW = NC * NS
LB = L // 8
BB = B // 128
EB = EMBED // 8

_mesh = plsc.VectorSubcoreMesh(
    core_axis_name="c", subcore_axis_name="s", num_cores=NC, num_subcores=NS
)


@functools.partial(
    pl.kernel,
    out_type=jax.ShapeDtypeStruct((L, EB, BB, 8, 128), jnp.float32),
    mesh=_mesh,
    scratch_types=[
        pltpu.VMEM((LB, 8, 128), jnp.int32),
        pltpu.VMEM((128, EMBED), jnp.float32),
        pltpu.VMEM((128, EMBED), jnp.float32),
        pltpu.VMEM((128, EMBED), jnp.float32),
        pltpu.VMEM((EMBED, 133), jnp.float32),
        pltpu.VMEM((EMBED, 133), jnp.float32),
        pltpu.SemaphoreType.DMA,
        pltpu.SemaphoreType.DMA,
        pltpu.SemaphoreType.DMA,
        pltpu.SemaphoreType.DMA,
        pltpu.SemaphoreType.DMA,
    ],
    compiler_params=pltpu.CompilerParams(
        use_tc_tiling_on_sc=False, needs_layout_passes=False),
)
def _gather_kernel(idx_hbm, table_hbm, out_hbm, idx_v, r0, r1, r2, t0, t1,
                   g0, g1, g2, s0, s1):
    rbuf = (r0, r1, r2)
    tbuf = (t0, t1)
    gsem = (g0, g1, g2)
    ssem = (s0, s1)
    wid = lax.axis_index("s") * NC + lax.axis_index("c")

    iota16 = lax.iota(jnp.int32, 16)
    evecs = [16 * m + iota16 for m in range(4)]

    def start_gather(a, s, p):
        pltpu.async_copy(table_hbm.at[idx_v.at[a, s]], rbuf[p], gsem[p])

    def start_gather_l(l, p):
        start_gather(l // 8, l % 8, p)

    def start_store(l, p):
        for eb in range(EB):
            pltpu.async_copy(
                tbuf[p].at[pl.ds(8 * eb, 8), pl.ds(0, 128)],
                out_hbm.at[l, eb, wid], ssem[p])

    def drain_gather(p):
        pltpu.make_async_copy(
            table_hbm.at[pl.ds(0, 128)], rbuf[p], gsem[p]).wait()

    def drain_store(p):
        for eb in range(EB):
            pltpu.make_async_copy(
                tbuf[p].at[pl.ds(8 * eb, 8), pl.ds(0, 128)],
                out_hbm.at[0, eb, wid], ssem[p]).wait()

    def transpose(p3, p2):
        @pl.loop(0, 128, step=8)
        def _t(j0):
            for dj in range(8):
                jvec = jnp.full((16,), 0, jnp.int32) + (j0 + dj)
                for m in range(4):
                    x = rbuf[p3][j0 + dj, pl.ds(16 * m, 16)]
                    plsc.store_scatter(tbuf[p2], [evecs[m], jvec], x)

    def body(l, p3, p2, issue_gather=True):
        if issue_gather:
            start_gather_l(l + 2, (p3 + 2) % 3)
        drain_gather(p3)

        @pl.when(l >= 2)
        def _():
            drain_store(p2)
        transpose(p3, p2)
        start_store(l, p2)

    pltpu.sync_copy(idx_hbm.at[:, wid], idx_v)

    start_gather(0, 0, 0)
    start_gather(0, 1, 1)

    @pl.loop(0, L - 2, step=6)
    def _steady(l0):
        for k in range(6):
            body(l0 + k, k % 3, k % 2)

    body(L - 2, (L - 2) % 3, 0, issue_gather=False)
    body(L - 1, (L - 1) % 3, 1, issue_gather=False)

    drain_store(0)
    drain_store(1)


@jax.jit
def kernel(indices, table):
    idx5 = (indices.astype(jnp.int32).T
            .reshape(LB, 8, BB, 128).transpose(0, 2, 1, 3)) * 2
    table_pad = jnp.pad(table, ((0, 0), (0, 64))).reshape(2 * 1000000, EMBED)
    out5 = _gather_kernel(idx5, table_pad)
    return out5.transpose(2, 4, 0, 1, 3).reshape(B, L, EMBED)


# speedup vs baseline: 1.0135x; 1.0135x over previous
import functools

import jax
import jax.numpy as jnp
from jax import lax
from jax.experimental import pallas as pl
from jax.experimental.pallas import tpu as pltpu
from jax.experimental.pallas import tpu_sc as plsc

B = 4096
L = 200
EMBED = 64
NC = 2
NS = 16
NW = NC * NS
LB = L // 8
BB = B // 128
EB = EMBED // 8

_mesh = plsc.VectorSubcoreMesh(
    core_axis_name="c", subcore_axis_name="s", num_cores=NC, num_subcores=NS
)


@functools.partial(
    pl.kernel,
    out_type=jax.ShapeDtypeStruct((L, EB, BB, 8, 128), jnp.float32),
    mesh=_mesh,
    scratch_types=[
        pltpu.VMEM((LB, 8, 128), jnp.int32),
        pltpu.VMEM((128, EMBED), jnp.float32),
        pltpu.VMEM((128, EMBED), jnp.float32),
        pltpu.VMEM((EMBED, 133), jnp.float32),
        pltpu.VMEM((EMBED, 133), jnp.float32),
        pltpu.SemaphoreType.DMA,
        pltpu.SemaphoreType.DMA,
        pltpu.SemaphoreType.DMA,
        pltpu.SemaphoreType.DMA,
    ],
    compiler_params=pltpu.CompilerParams(
        use_tc_tiling_on_sc=False, needs_layout_passes=False),
)
def _gather_kernel(idx_hbm, table_hbm, out_hbm, idx_v, r0, r1, t0, t1,
                   g0, g1, s0, s1):
    rbuf = (r0, r1)
    tbuf = (t0, t1)
    gsem = (g0, g1)
    ssem = (s0, s1)
    wid = lax.axis_index("s") * NC + lax.axis_index("c")

    iota16 = lax.iota(jnp.int32, 16)
    evecs = [16 * m + iota16 for m in range(4)]

    def start_gather(a, s, p):
        pltpu.async_copy(table_hbm.at[idx_v.at[a, s]], rbuf[p], gsem[p])

    def start_gather_l(l, p):
        start_gather(l // 8, l % 8, p)

    def start_store(l, p):
        for eb in range(EB):
            pltpu.async_copy(
                tbuf[p].at[pl.ds(8 * eb, 8), pl.ds(0, 128)],
                out_hbm.at[l, eb, wid], ssem[p])

    def drain_gather(p):
        pltpu.make_async_copy(
            table_hbm.at[pl.ds(0, 128)], rbuf[p], gsem[p]).wait()

    def drain_store(p):
        for eb in range(EB):
            pltpu.make_async_copy(
                tbuf[p].at[pl.ds(8 * eb, 8), pl.ds(0, 128)],
                out_hbm.at[0, eb, wid], ssem[p]).wait()

    def transpose(p):
        @pl.loop(0, 128, step=8)
        def _t(j0):
            for dj in range(8):
                jvec = jnp.full((16,), 0, jnp.int32) + (j0 + dj)
                for m in range(4):
                    x = rbuf[p][j0 + dj, pl.ds(16 * m, 16)]
                    plsc.store_scatter(tbuf[p], [evecs[m], jvec], x)

    def body(l, p):
        @pl.when(l + 1 < L)
        def _():
            start_gather_l(l + 1, 1 - p)
        drain_gather(p)

        @pl.when(l >= 2)
        def _():
            drain_store(p)
        transpose(p)
        start_store(l, p)

    pltpu.sync_copy(idx_hbm.at[:, wid], idx_v)

    start_gather(0, 0, 0)

    @pl.loop(0, L, step=2)
    def _steady(l0):
        for k in range(2):
            body(l0 + k, k)

    drain_store(0)
    drain_store(1)


@jax.jit
def kernel(indices, table):
    idx5 = (indices.astype(jnp.int32).T
            .reshape(LB, 8, BB, 128).transpose(0, 2, 1, 3)) * 2
    table_pad = jnp.pad(table, ((0, 0), (0, 64))).reshape(2 * 1000000, EMBED)
    out5 = _gather_kernel(idx5, table_pad)
    return out5.transpose(2, 4, 0, 1, 3).reshape(B, L, EMBED)
